# Initial kernel scaffold; baseline (speedup 1.0000x reference)
import jax
import jax.numpy as jnp
from jax.experimental import pallas as pl


def kernel(x, table, W, b):
    raise NotImplementedError("write your pallas kernel here")


# speedup vs baseline: 2.2588x; 2.2588x over previous
import functools

import jax
import jax.numpy as jnp
from jax import lax
from jax.experimental import pallas as pl
from jax.experimental.pallas import tpu as pltpu
from jax.experimental.pallas import tpu_sc as plsc

VOCAB = 1000000
EMBED_DIM = 32
BATCH = 4096
SEQ = 200


_ROWS_PER_BLOCK = 8000


def _proj_body(t_ref, w_ref, bias_ref, o_ref):
    w = w_ref[0, :]
    t = t_ref[...]
    s = jnp.sum(t * w[None, :], axis=1)
    s = (s + bias_ref[0, 0]) * (1.0 / SEQ)
    o_ref[...] = s.reshape(o_ref.shape)


def _project_table(table, W, b):
    nblk = VOCAB // _ROWS_PER_BLOCK
    bias = b.reshape(1, 1)
    out = pl.pallas_call(
        _proj_body,
        grid=(nblk,),
        in_specs=[
            pl.BlockSpec((_ROWS_PER_BLOCK, EMBED_DIM), lambda i: (i, 0)),
            pl.BlockSpec((1, EMBED_DIM), lambda i: (0, 0)),
            pl.BlockSpec(memory_space=pltpu.SMEM),
        ],
        out_specs=pl.BlockSpec((8, _ROWS_PER_BLOCK // 8), lambda i: (i, 0)),
        out_shape=jax.ShapeDtypeStruct((nblk * 8, _ROWS_PER_BLOCK // 8),
                                       jnp.float32),
    )(table, W, bias)
    return out.reshape(VOCAB)


_NC, _NS = 2, 16
_NW = _NC * _NS
_BPW = BATCH // _NW
_IPW = _BPW * SEQ
_GCHUNK = 128
_GWAVE = 8


def _tr_body(x_ref, o_ref):
    o_ref[...] = x_ref[...].T.reshape(o_ref.shape)


def _transpose_x(x):
    out = pl.pallas_call(
        _tr_body,
        grid=(_NW,),
        in_specs=[pl.BlockSpec((_BPW, SEQ), lambda i: (i, 0))],
        out_specs=pl.BlockSpec((1, SEQ, _BPW), lambda i: (i, 0, 0)),
        out_shape=jax.ShapeDtypeStruct((_NW, SEQ, _BPW), jnp.int32),
    )(x)
    return out.reshape(_NW, _IPW)


def _pool_body(x_hbm, p_hbm, out_hbm, xv, vals, accv, sem):
    w = lax.axis_index("s") * _NC + lax.axis_index("c")
    pltpu.sync_copy(x_hbm.at[w], xv)

    def gather_wave(g, _):
        base = pl.multiple_of(g * (_GCHUNK * _GWAVE), 8)
        for u in range(_GWAVE):
            off = pl.multiple_of(base + u * _GCHUNK, 8)
            pltpu.async_copy(
                p_hbm.at[xv.at[pl.ds(off, _GCHUNK)]],
                vals.at[pl.ds(off, _GCHUNK)],
                sem,
            )
        for u in range(_GWAVE):
            off = pl.multiple_of(base + u * _GCHUNK, 8)
            pltpu.make_async_copy(
                p_hbm.at[xv.at[pl.ds(off, _GCHUNK)]],
                vals.at[pl.ds(off, _GCHUNK)],
                sem,
            ).wait()
        return _

    lax.fori_loop(0, _IPW // (_GCHUNK * _GWAVE), gather_wave, 0)

    nchunk = _BPW // 16

    def body(j, accs):
        base = j * _BPW
        return tuple(
            accs[c] + vals[pl.ds(base + c * 16, 16)]
            for c in range(nchunk)
        )

    zero = jnp.zeros((16,), jnp.float32)
    accs = lax.fori_loop(0, SEQ, body, tuple(zero for _ in range(nchunk)))

    for c in range(nchunk):
        z = accs[c]
        accv[pl.ds(c * 16, 16)] = 1.0 / (1.0 + jnp.exp(-z))
    pltpu.sync_copy(accv, out_hbm.at[w])


@functools.partial(
    pl.kernel,
    mesh=plsc.VectorSubcoreMesh(core_axis_name="c", subcore_axis_name="s"),
    out_type=jax.ShapeDtypeStruct((_NW, _BPW), jnp.float32),
    scratch_types=[
        pltpu.VMEM((_IPW,), jnp.int32),
        pltpu.VMEM((_IPW,), jnp.float32),
        pltpu.VMEM((_BPW,), jnp.float32),
        pltpu.SemaphoreType.DMA,
    ],
)
def _pool_kernel(x_hbm, p_hbm, out_hbm, xv, vals, accv, sem):
    _pool_body(x_hbm, p_hbm, out_hbm, xv, vals, accv, sem)


def kernel(x, table, W, b):
    p = _project_table(table, W, b)
    xt = _transpose_x(x)
    out = _pool_kernel(xt, p)
    return out.reshape(BATCH, 1)


# speedup vs baseline: 6.8924x; 3.0514x over previous
import functools

import jax
import jax.numpy as jnp
from jax import lax
from jax.experimental import pallas as pl
from jax.experimental.pallas import tpu as pltpu
from jax.experimental.pallas import tpu_sc as plsc

VOCAB = 1000000
EMBED_DIM = 32
BATCH = 4096
SEQ = 200


_ROWS_PER_BLOCK = 8192
_NPBLK = 123


def _proj_body(t_ref, w_ref, bias_ref, o_ref):
    t = t_ref[...]
    s = jnp.dot(w_ref[...], t,
                preferred_element_type=jnp.float32)[0]
    s = (s + bias_ref[0, 0]) * (1.0 / SEQ)
    o_ref[...] = s.reshape(o_ref.shape)


def _project_table(table, W, b):
    t_t = table.T
    nblk = _NPBLK
    bias = b.reshape(1, 1)
    out = pl.pallas_call(
        _proj_body,
        grid=(nblk,),
        in_specs=[
            pl.BlockSpec((EMBED_DIM, _ROWS_PER_BLOCK), lambda i: (0, i)),
            pl.BlockSpec((1, EMBED_DIM), lambda i: (0, 0)),
            pl.BlockSpec(memory_space=pltpu.SMEM),
        ],
        out_specs=pl.BlockSpec((8, _ROWS_PER_BLOCK // 8), lambda i: (i, 0)),
        out_shape=jax.ShapeDtypeStruct((nblk * 8, _ROWS_PER_BLOCK // 8),
                                       jnp.float32),
    )(t_t, W, bias)
    return out.reshape(nblk * _ROWS_PER_BLOCK)[:VOCAB]


_NC, _NS = 2, 16
_NW = _NC * _NS
_BPW = BATCH // _NW
_IPW = _BPW * SEQ
_GCHUNK = 128
_GWAVE = 8


def _tr_body(x_ref, o_ref):
    o_ref[...] = x_ref[...].T.reshape(o_ref.shape)


def _transpose_x(x):
    out = pl.pallas_call(
        _tr_body,
        grid=(_NW,),
        in_specs=[pl.BlockSpec((_BPW, SEQ), lambda i: (i, 0))],
        out_specs=pl.BlockSpec((1, SEQ, _BPW), lambda i: (i, 0, 0)),
        out_shape=jax.ShapeDtypeStruct((_NW, SEQ, _BPW), jnp.int32),
    )(x)
    return out.reshape(_NW, _IPW)


def _pool_body(x_hbm, p_hbm, out_hbm, xv, vals, accv, sem):
    w = lax.axis_index("s") * _NC + lax.axis_index("c")
    pltpu.sync_copy(x_hbm.at[w], xv)

    def gather_wave(g, _):
        base = pl.multiple_of(g * (_GCHUNK * _GWAVE), 8)
        for u in range(_GWAVE):
            off = pl.multiple_of(base + u * _GCHUNK, 8)
            pltpu.async_copy(
                p_hbm.at[xv.at[pl.ds(off, _GCHUNK)]],
                vals.at[pl.ds(off, _GCHUNK)],
                sem,
            )
        for u in range(_GWAVE):
            off = pl.multiple_of(base + u * _GCHUNK, 8)
            pltpu.make_async_copy(
                p_hbm.at[xv.at[pl.ds(off, _GCHUNK)]],
                vals.at[pl.ds(off, _GCHUNK)],
                sem,
            ).wait()
        return _

    lax.fori_loop(0, _IPW // (_GCHUNK * _GWAVE), gather_wave, 0)

    nchunk = _BPW // 16

    def body(j, accs):
        base = j * _BPW
        return tuple(
            accs[c] + vals[pl.ds(base + c * 16, 16)]
            for c in range(nchunk)
        )

    zero = jnp.zeros((16,), jnp.float32)
    accs = lax.fori_loop(0, SEQ, body, tuple(zero for _ in range(nchunk)))

    for c in range(nchunk):
        z = accs[c]
        accv[pl.ds(c * 16, 16)] = 1.0 / (1.0 + jnp.exp(-z))
    pltpu.sync_copy(accv, out_hbm.at[w])


@functools.partial(
    pl.kernel,
    mesh=plsc.VectorSubcoreMesh(core_axis_name="c", subcore_axis_name="s"),
    out_type=jax.ShapeDtypeStruct((_NW, _BPW), jnp.float32),
    scratch_types=[
        pltpu.VMEM((_IPW,), jnp.int32),
        pltpu.VMEM((_IPW,), jnp.float32),
        pltpu.VMEM((_BPW,), jnp.float32),
        pltpu.SemaphoreType.DMA,
    ],
)
def _pool_kernel(x_hbm, p_hbm, out_hbm, xv, vals, accv, sem):
    _pool_body(x_hbm, p_hbm, out_hbm, xv, vals, accv, sem)


def kernel(x, table, W, b):
    p = _project_table(table, W, b)
    xt = _transpose_x(x)
    out = _pool_kernel(xt, p)
    return out.reshape(BATCH, 1)


# speedup vs baseline: 9.8313x; 1.4264x over previous
import functools

import jax
import jax.numpy as jnp
from jax import lax
from jax.experimental import pallas as pl
from jax.experimental.pallas import tpu as pltpu
from jax.experimental.pallas import tpu_sc as plsc

VOCAB = 1000000
EMBED_DIM = 32
BATCH = 4096
SEQ = 200


_ROWS_PER_BLOCK = 32768
_NPBLK = 31


def _proj_body(t_ref, w_ref, bias_ref, o_ref):
    t = t_ref[...]
    s = jnp.dot(w_ref[...], t,
                preferred_element_type=jnp.float32)[0]
    s = (s + bias_ref[0, 0]) * (1.0 / SEQ)
    o_ref[...] = s.reshape(o_ref.shape)


def _project_table(table, W, b):
    t_t = table.T
    nblk = _NPBLK
    bias = b.reshape(1, 1)
    out = pl.pallas_call(
        _proj_body,
        grid=(nblk,),
        in_specs=[
            pl.BlockSpec((EMBED_DIM, _ROWS_PER_BLOCK), lambda i: (0, i)),
            pl.BlockSpec((1, EMBED_DIM), lambda i: (0, 0)),
            pl.BlockSpec(memory_space=pltpu.SMEM),
        ],
        out_specs=pl.BlockSpec((8, _ROWS_PER_BLOCK // 8), lambda i: (i, 0)),
        out_shape=jax.ShapeDtypeStruct((nblk * 8, _ROWS_PER_BLOCK // 8),
                                       jnp.float32),
    )(t_t, W, bias)
    return out.reshape(nblk * _ROWS_PER_BLOCK)[:VOCAB]


_NC, _NS = 2, 16
_NW = _NC * _NS
_BPW = BATCH // _NW
_IPW = _BPW * SEQ
_GCHUNK = 128
_GWAVE = 8


def _tr_body(x_ref, o_ref):
    o_ref[...] = x_ref[...].T.reshape(o_ref.shape)


def _transpose_x(x):
    out = pl.pallas_call(
        _tr_body,
        grid=(_NW,),
        in_specs=[pl.BlockSpec((_BPW, SEQ), lambda i: (i, 0))],
        out_specs=pl.BlockSpec((1, SEQ, _BPW), lambda i: (i, 0, 0)),
        out_shape=jax.ShapeDtypeStruct((_NW, SEQ, _BPW), jnp.int32),
    )(x)
    return out.reshape(_NW, _IPW)


def _pool_body(x_hbm, p_hbm, out_hbm, xv, vals, accv, sem):
    w = lax.axis_index("s") * _NC + lax.axis_index("c")
    pltpu.sync_copy(x_hbm.at[w], xv)

    nwave = _IPW // (_GCHUNK * _GWAVE)
    wbytes = _GCHUNK * _GWAVE

    def fire(base):
        for u in range(_GWAVE):
            off = pl.multiple_of(base + u * _GCHUNK, 8)
            pltpu.async_copy(
                p_hbm.at[xv.at[pl.ds(off, _GCHUNK)]],
                vals.at[pl.ds(off, _GCHUNK)],
                sem,
            )

    def drain(base):
        for u in range(_GWAVE):
            off = pl.multiple_of(base + u * _GCHUNK, 8)
            pltpu.make_async_copy(
                p_hbm.at[xv.at[pl.ds(off, _GCHUNK)]],
                vals.at[pl.ds(off, _GCHUNK)],
                sem,
            ).wait()

    fire(0)

    def gather_wave(g, _):
        fire(pl.multiple_of((g + 1) * wbytes, 8))
        drain(pl.multiple_of(g * wbytes, 8))
        return _

    lax.fori_loop(0, nwave - 1, gather_wave, 0)
    drain(pl.multiple_of((nwave - 1) * wbytes, 8))

    nchunk = _BPW // 16

    def body(j, accs):
        base = j * _BPW
        return tuple(
            accs[c] + vals[pl.ds(base + c * 16, 16)]
            for c in range(nchunk)
        )

    zero = jnp.zeros((16,), jnp.float32)
    accs = lax.fori_loop(0, SEQ, body, tuple(zero for _ in range(nchunk)))

    for c in range(nchunk):
        z = accs[c]
        accv[pl.ds(c * 16, 16)] = 1.0 / (1.0 + jnp.exp(-z))
    pltpu.sync_copy(accv, out_hbm.at[w])


@functools.partial(
    pl.kernel,
    mesh=plsc.VectorSubcoreMesh(core_axis_name="c", subcore_axis_name="s"),
    out_type=jax.ShapeDtypeStruct((_NW, _BPW), jnp.float32),
    scratch_types=[
        pltpu.VMEM((_IPW,), jnp.int32),
        pltpu.VMEM((_IPW,), jnp.float32),
        pltpu.VMEM((_BPW,), jnp.float32),
        pltpu.SemaphoreType.DMA,
    ],
)
def _pool_kernel(x_hbm, p_hbm, out_hbm, xv, vals, accv, sem):
    _pool_body(x_hbm, p_hbm, out_hbm, xv, vals, accv, sem)


def kernel(x, table, W, b):
    p = _project_table(table, W, b)
    xt = _transpose_x(x)
    out = _pool_kernel(xt, p)
    return out.reshape(BATCH, 1)


# speedup vs baseline: 10.4998x; 1.0680x over previous
import functools

import jax
import jax.numpy as jnp
from jax import lax
from jax.experimental import pallas as pl
from jax.experimental.pallas import tpu as pltpu
from jax.experimental.pallas import tpu_sc as plsc

VOCAB = 1000000
EMBED_DIM = 32
BATCH = 4096
SEQ = 200


_ROWS_PER_BLOCK = 65536
_NPBLK = 16


def _proj_body(t_ref, w_ref, bias_ref, o_ref):
    t = t_ref[...]
    s = jnp.dot(w_ref[...], t,
                preferred_element_type=jnp.float32)[0]
    s = (s + bias_ref[0, 0]) * (1.0 / SEQ)
    o_ref[...] = s.reshape(o_ref.shape)


def _project_table(table, W, b):
    t_t = table.T
    nblk = _NPBLK
    bias = b.reshape(1, 1)
    out = pl.pallas_call(
        _proj_body,
        grid=(nblk,),
        in_specs=[
            pl.BlockSpec((EMBED_DIM, _ROWS_PER_BLOCK), lambda i: (0, i)),
            pl.BlockSpec((1, EMBED_DIM), lambda i: (0, 0)),
            pl.BlockSpec(memory_space=pltpu.SMEM),
        ],
        out_specs=pl.BlockSpec((8, _ROWS_PER_BLOCK // 8), lambda i: (i, 0)),
        out_shape=jax.ShapeDtypeStruct((nblk * 8, _ROWS_PER_BLOCK // 8),
                                       jnp.float32),
    )(t_t, W, bias)
    return out.reshape(nblk * _ROWS_PER_BLOCK)[:VOCAB]


_NC, _NS = 2, 16
_NW = _NC * _NS
_BPW = BATCH // _NW
_IPW = _BPW * SEQ
_GCHUNK = 512
_GWAVE = 5


def _tr_body(x_ref, o_ref):
    o_ref[...] = x_ref[...].T.reshape(o_ref.shape)


def _transpose_x(x):
    out = pl.pallas_call(
        _tr_body,
        grid=(_NW,),
        in_specs=[pl.BlockSpec((_BPW, SEQ), lambda i: (i, 0))],
        out_specs=pl.BlockSpec((1, SEQ, _BPW), lambda i: (i, 0, 0)),
        out_shape=jax.ShapeDtypeStruct((_NW, SEQ, _BPW), jnp.int32),
    )(x)
    return out.reshape(_NW, _IPW)


def _pool_body(x_hbm, p_hbm, out_hbm, xv, vals, accv, sem):
    w = lax.axis_index("s") * _NC + lax.axis_index("c")
    pltpu.sync_copy(x_hbm.at[w], xv)

    nwave = _IPW // (_GCHUNK * _GWAVE)
    wbytes = _GCHUNK * _GWAVE

    def fire(base):
        for u in range(_GWAVE):
            off = pl.multiple_of(base + u * _GCHUNK, 8)
            pltpu.async_copy(
                p_hbm.at[xv.at[pl.ds(off, _GCHUNK)]],
                vals.at[pl.ds(off, _GCHUNK)],
                sem,
            )

    def drain(base):
        for u in range(_GWAVE):
            off = pl.multiple_of(base + u * _GCHUNK, 8)
            pltpu.make_async_copy(
                p_hbm.at[xv.at[pl.ds(off, _GCHUNK)]],
                vals.at[pl.ds(off, _GCHUNK)],
                sem,
            ).wait()

    fire(0)

    def gather_wave(g, _):
        fire(pl.multiple_of((g + 1) * wbytes, 8))
        drain(pl.multiple_of(g * wbytes, 8))
        return _

    lax.fori_loop(0, nwave - 1, gather_wave, 0)
    drain(pl.multiple_of((nwave - 1) * wbytes, 8))

    nchunk = _BPW // 16

    def body(j, accs):
        base = j * _BPW
        return tuple(
            accs[c] + vals[pl.ds(base + c * 16, 16)]
            for c in range(nchunk)
        )

    zero = jnp.zeros((16,), jnp.float32)
    accs = lax.fori_loop(0, SEQ, body, tuple(zero for _ in range(nchunk)))

    for c in range(nchunk):
        z = accs[c]
        accv[pl.ds(c * 16, 16)] = 1.0 / (1.0 + jnp.exp(-z))
    pltpu.sync_copy(accv, out_hbm.at[w])


@functools.partial(
    pl.kernel,
    mesh=plsc.VectorSubcoreMesh(core_axis_name="c", subcore_axis_name="s"),
    out_type=jax.ShapeDtypeStruct((_NW, _BPW), jnp.float32),
    scratch_types=[
        pltpu.VMEM((_IPW,), jnp.int32),
        pltpu.VMEM((_IPW,), jnp.float32),
        pltpu.VMEM((_BPW,), jnp.float32),
        pltpu.SemaphoreType.DMA,
    ],
)
def _pool_kernel(x_hbm, p_hbm, out_hbm, xv, vals, accv, sem):
    _pool_body(x_hbm, p_hbm, out_hbm, xv, vals, accv, sem)


def kernel(x, table, W, b):
    p = _project_table(table, W, b)
    xt = _transpose_x(x)
    out = _pool_kernel(xt, p)
    return out.reshape(BATCH, 1)


# speedup vs baseline: 10.8103x; 1.0296x over previous
import functools

import jax
import jax.numpy as jnp
from jax import lax
from jax.experimental import pallas as pl
from jax.experimental.pallas import tpu as pltpu
from jax.experimental.pallas import tpu_sc as plsc

VOCAB = 1000000
EMBED_DIM = 32
BATCH = 4096
SEQ = 200


_ROWS_PER_BLOCK = 65536
_NPBLK = 16


def _proj_body(t_ref, w_ref, bias_ref, o_ref):
    t = t_ref[...]
    s = jnp.dot(w_ref[...], t,
                preferred_element_type=jnp.float32)[0]
    s = (s + bias_ref[0, 0]) * (1.0 / SEQ)
    o_ref[...] = s.reshape(o_ref.shape)


def _project_table(table, W, b):
    t_t = table.T
    nblk = _NPBLK
    bias = b.reshape(1, 1)
    out = pl.pallas_call(
        _proj_body,
        grid=(nblk,),
        in_specs=[
            pl.BlockSpec((EMBED_DIM, _ROWS_PER_BLOCK), lambda i: (0, i)),
            pl.BlockSpec((1, EMBED_DIM), lambda i: (0, 0)),
            pl.BlockSpec(memory_space=pltpu.SMEM),
        ],
        out_specs=pl.BlockSpec((8, _ROWS_PER_BLOCK // 8), lambda i: (i, 0)),
        out_shape=jax.ShapeDtypeStruct((nblk * 8, _ROWS_PER_BLOCK // 8),
                                       jnp.float32),
    )(t_t, W, bias)
    return out.reshape(nblk * _ROWS_PER_BLOCK)[:VOCAB]


_NC, _NS = 2, 16
_NW = _NC * _NS
_BPW = BATCH // _NW
_IPW = _BPW * SEQ
_GCHUNK = 1024
_GWAVE = 5


def _tr_body(x_ref, o_ref):
    o_ref[...] = x_ref[...].T.reshape(o_ref.shape)


def _transpose_x(x):
    out = pl.pallas_call(
        _tr_body,
        grid=(_NW,),
        in_specs=[pl.BlockSpec((_BPW, SEQ), lambda i: (i, 0))],
        out_specs=pl.BlockSpec((1, SEQ, _BPW), lambda i: (i, 0, 0)),
        out_shape=jax.ShapeDtypeStruct((_NW, SEQ, _BPW), jnp.int32),
    )(x)
    return out.reshape(_NW, _IPW)


def _pool_body(x_hbm, p_hbm, out_hbm, xv, vals, accv, sem):
    w = lax.axis_index("s") * _NC + lax.axis_index("c")
    pltpu.sync_copy(x_hbm.at[w], xv)

    nwave = _IPW // (_GCHUNK * _GWAVE)
    wbytes = _GCHUNK * _GWAVE

    def fire(base):
        for u in range(_GWAVE):
            off = pl.multiple_of(base + u * _GCHUNK, 8)
            pltpu.async_copy(
                p_hbm.at[xv.at[pl.ds(off, _GCHUNK)]],
                vals.at[pl.ds(off, _GCHUNK)],
                sem,
            )

    def drain(base):
        for u in range(_GWAVE):
            off = pl.multiple_of(base + u * _GCHUNK, 8)
            pltpu.make_async_copy(
                p_hbm.at[xv.at[pl.ds(off, _GCHUNK)]],
                vals.at[pl.ds(off, _GCHUNK)],
                sem,
            ).wait()

    fire(0)

    def gather_wave(g, _):
        fire(pl.multiple_of((g + 1) * wbytes, 8))
        drain(pl.multiple_of(g * wbytes, 8))
        return _

    lax.fori_loop(0, nwave - 1, gather_wave, 0)
    drain(pl.multiple_of((nwave - 1) * wbytes, 8))

    nchunk = _BPW // 16

    def body(j, accs):
        base = j * _BPW
        return tuple(
            accs[c] + vals[pl.ds(base + c * 16, 16)]
            for c in range(nchunk)
        )

    zero = jnp.zeros((16,), jnp.float32)
    accs = lax.fori_loop(0, SEQ, body, tuple(zero for _ in range(nchunk)))

    for c in range(nchunk):
        z = accs[c]
        accv[pl.ds(c * 16, 16)] = 1.0 / (1.0 + jnp.exp(-z))
    pltpu.sync_copy(accv, out_hbm.at[w])


@functools.partial(
    pl.kernel,
    mesh=plsc.VectorSubcoreMesh(core_axis_name="c", subcore_axis_name="s"),
    out_type=jax.ShapeDtypeStruct((_NW, _BPW), jnp.float32),
    scratch_types=[
        pltpu.VMEM((_IPW,), jnp.int32),
        pltpu.VMEM((_IPW,), jnp.float32),
        pltpu.VMEM((_BPW,), jnp.float32),
        pltpu.SemaphoreType.DMA,
    ],
)
def _pool_kernel(x_hbm, p_hbm, out_hbm, xv, vals, accv, sem):
    _pool_body(x_hbm, p_hbm, out_hbm, xv, vals, accv, sem)


def kernel(x, table, W, b):
    p = _project_table(table, W, b)
    xt = _transpose_x(x)
    out = _pool_kernel(xt, p)
    return out.reshape(BATCH, 1)


# speedup vs baseline: 12.4880x; 1.1552x over previous
import functools

import jax
import jax.numpy as jnp
from jax import lax
from jax.experimental import pallas as pl
from jax.experimental.pallas import tpu as pltpu
from jax.experimental.pallas import tpu_sc as plsc

VOCAB = 1000000
EMBED_DIM = 32
BATCH = 4096
SEQ = 200


_ROWS_PER_BLOCK = 65536
_NPBLK = 16


def _proj_body(t_ref, w_ref, bias_ref, o_ref):
    t = t_ref[...]
    s = jnp.dot(w_ref[...], t,
                preferred_element_type=jnp.float32)[0]
    s = (s + bias_ref[0, 0]) * (1.0 / SEQ)
    o_ref[...] = s.reshape(o_ref.shape)


def _project_table(table, W, b):
    t_t = table.T
    nblk = _NPBLK
    bias = b.reshape(1, 1)
    out = pl.pallas_call(
        _proj_body,
        grid=(nblk,),
        in_specs=[
            pl.BlockSpec((EMBED_DIM, _ROWS_PER_BLOCK), lambda i: (0, i)),
            pl.BlockSpec((1, EMBED_DIM), lambda i: (0, 0)),
            pl.BlockSpec(memory_space=pltpu.SMEM),
        ],
        out_specs=pl.BlockSpec((8, _ROWS_PER_BLOCK // 8), lambda i: (i, 0)),
        out_shape=jax.ShapeDtypeStruct((nblk * 8, _ROWS_PER_BLOCK // 8),
                                       jnp.float32),
    )(t_t, W, bias)
    return out.reshape(nblk * _ROWS_PER_BLOCK)[:VOCAB]


_NC, _NS = 2, 16
_NW = _NC * _NS
_BPW = BATCH // _NW
_IPW = _BPW * SEQ
_GCHUNK = 1024
_GWAVE = 5


def _tr_body(x_ref, o_ref):
    o_ref[...] = x_ref[...].T.reshape(o_ref.shape)


def _transpose_x(x):
    out = pl.pallas_call(
        _tr_body,
        grid=(_NW,),
        in_specs=[pl.BlockSpec((_BPW, SEQ), lambda i: (i, 0))],
        out_specs=pl.BlockSpec((1, SEQ, _BPW), lambda i: (i, 0, 0)),
        out_shape=jax.ShapeDtypeStruct((_NW, SEQ, _BPW), jnp.int32),
    )(x)
    return out.reshape(_NW, _IPW)


_PAIR = 2 * SEQ
_WROWS = 16
_NWAVE = _BPW // _WROWS
_WELEMS = _WROWS * SEQ


def _pool_body(x_hbm, p_hbm, out_hbm, xv, vals, accv, sem):
    w = lax.axis_index("s") * _NC + lax.axis_index("c")
    pltpu.sync_copy(x_hbm.at[w], xv)

    def fire(g):
        for u in range(_NWAVE):
            off = g * _WELEMS + u * _PAIR
            pltpu.async_copy(
                p_hbm.at[xv.at[pl.ds(off, _PAIR)]],
                vals.at[pl.ds(off, _PAIR)],
                sem,
            )

    def drain(g):
        for u in range(_NWAVE):
            off = g * _WELEMS + u * _PAIR
            pltpu.make_async_copy(
                p_hbm.at[xv.at[pl.ds(off, _PAIR)]],
                vals.at[pl.ds(off, _PAIR)],
                sem,
            ).wait()

    lane = lax.iota(jnp.int32, 16)
    first_half = lane < 8
    lane_is = [lane == i for i in range(_WROWS)]
    bfly = [(lane ^ s)[:, None] for s in (8, 4, 2, 1)]
    _gd = lax.GatherDimensionNumbers(
        offset_dims=(), collapsed_slice_dims=(0,), start_index_map=(0,))

    def lanesum(v):
        for perm in bfly:
            v = v + lax.gather(v, perm, _gd, slice_sizes=(1,),
                               mode=lax.GatherScatterMode.PROMISE_IN_BOUNDS)
        return v

    fire(0)
    for g in range(_NWAVE):
        if g + 1 < _NWAVE:
            fire(g + 1)
        drain(g)
        acc = jnp.zeros((16,), jnp.float32)
        for u in range(_NWAVE):
            base = g * _WELEMS + u * _PAIR
            vs = [vals[pl.ds(base + 16 * k, 16)] for k in range(25)]
            sa = vs[0]
            for k in range(1, 12):
                sa = sa + vs[k]
            sa = sa + jnp.where(first_half, vs[12], 0.0)
            sb = jnp.where(first_half, 0.0, vs[12])
            for k in range(13, 25):
                sb = sb + vs[k]
            acc = jnp.where(lane_is[2 * u], lanesum(sa), acc)
            acc = jnp.where(lane_is[2 * u + 1], lanesum(sb), acc)
        accv[pl.ds(g * 16, 16)] = 1.0 / (1.0 + jnp.exp(-acc))

    pltpu.sync_copy(accv, out_hbm.at[w])


@functools.partial(
    pl.kernel,
    mesh=plsc.VectorSubcoreMesh(core_axis_name="c", subcore_axis_name="s"),
    out_type=jax.ShapeDtypeStruct((_NW, _BPW), jnp.float32),
    scratch_types=[
        pltpu.VMEM((_IPW,), jnp.int32),
        pltpu.VMEM((_IPW,), jnp.float32),
        pltpu.VMEM((_BPW,), jnp.float32),
        pltpu.SemaphoreType.DMA,
    ],
)
def _pool_kernel(x_hbm, p_hbm, out_hbm, xv, vals, accv, sem):
    _pool_body(x_hbm, p_hbm, out_hbm, xv, vals, accv, sem)


def kernel(x, table, W, b):
    p = _project_table(table, W, b)
    xr = x.reshape(_NW, _IPW)
    out = _pool_kernel(xr, p)
    return out.reshape(BATCH, 1)


# speedup vs baseline: 13.0756x; 1.0471x over previous
import functools

import jax
import jax.numpy as jnp
from jax import lax
from jax.experimental import pallas as pl
from jax.experimental.pallas import tpu as pltpu
from jax.experimental.pallas import tpu_sc as plsc

VOCAB = 1000000
EMBED_DIM = 32
BATCH = 4096
SEQ = 200


_ROWS_PER_BLOCK = 65536
_NPBLK = 16


def _proj_body(t_ref, w_ref, bias_ref, o_ref):
    t = t_ref[...]
    s = jnp.dot(w_ref[...], t,
                preferred_element_type=jnp.float32)[0]
    o_ref[...] = (s + bias_ref[0, 0]) * (1.0 / SEQ)


def _project_table(table, W, b):
    t_t = table.T
    nblk = _NPBLK
    bias = b.reshape(1, 1)
    out = pl.pallas_call(
        _proj_body,
        grid=(nblk,),
        in_specs=[
            pl.BlockSpec((EMBED_DIM, _ROWS_PER_BLOCK), lambda i: (0, i)),
            pl.BlockSpec((1, EMBED_DIM), lambda i: (0, 0)),
            pl.BlockSpec(memory_space=pltpu.SMEM),
        ],
        out_specs=pl.BlockSpec((_ROWS_PER_BLOCK,), lambda i: (i,)),
        out_shape=jax.ShapeDtypeStruct((nblk * _ROWS_PER_BLOCK,),
                                       jnp.float32),
    )(t_t, W, bias)
    return out


_NC, _NS = 2, 16
_NW = _NC * _NS
_BPW = BATCH // _NW
_IPW = _BPW * SEQ
_GCHUNK = 1024
_GWAVE = 5


def _tr_body(x_ref, o_ref):
    o_ref[...] = x_ref[...].T.reshape(o_ref.shape)


def _transpose_x(x):
    out = pl.pallas_call(
        _tr_body,
        grid=(_NW,),
        in_specs=[pl.BlockSpec((_BPW, SEQ), lambda i: (i, 0))],
        out_specs=pl.BlockSpec((1, SEQ, _BPW), lambda i: (i, 0, 0)),
        out_shape=jax.ShapeDtypeStruct((_NW, SEQ, _BPW), jnp.int32),
    )(x)
    return out.reshape(_NW, _IPW)


_PAIR = 2 * SEQ
_WROWS = 16
_NWAVE = _BPW // _WROWS
_WELEMS = _WROWS * SEQ


def _pool_body(x_hbm, p_hbm, out_hbm, xv, vals, accv, sem):
    w = lax.axis_index("s") * _NC + lax.axis_index("c")
    pltpu.sync_copy(x_hbm.at[w], xv)

    def fire(g):
        for u in range(_NWAVE):
            off = g * _WELEMS + u * _PAIR
            pltpu.async_copy(
                p_hbm.at[xv.at[pl.ds(off, _PAIR)]],
                vals.at[pl.ds(off, _PAIR)],
                sem,
            )

    def drain(g):
        for u in range(_NWAVE):
            off = g * _WELEMS + u * _PAIR
            pltpu.make_async_copy(
                p_hbm.at[xv.at[pl.ds(off, _PAIR)]],
                vals.at[pl.ds(off, _PAIR)],
                sem,
            ).wait()

    lane = lax.iota(jnp.int32, 16)
    first_half = lane < 8
    lane_is = [lane == i for i in range(_WROWS)]
    bfly = [(lane ^ s)[:, None] for s in (8, 4, 2, 1)]
    _gd = lax.GatherDimensionNumbers(
        offset_dims=(), collapsed_slice_dims=(0,), start_index_map=(0,))

    def lanesum(v):
        for perm in bfly:
            v = v + lax.gather(v, perm, _gd, slice_sizes=(1,),
                               mode=lax.GatherScatterMode.PROMISE_IN_BOUNDS)
        return v

    fire(0)
    for g in range(_NWAVE):
        if g + 1 < _NWAVE:
            fire(g + 1)
        drain(g)
        acc = jnp.zeros((16,), jnp.float32)
        for u in range(_NWAVE):
            base = g * _WELEMS + u * _PAIR
            vs = [vals[pl.ds(base + 16 * k, 16)] for k in range(25)]
            sa = vs[0]
            for k in range(1, 12):
                sa = sa + vs[k]
            sa = sa + jnp.where(first_half, vs[12], 0.0)
            sb = jnp.where(first_half, 0.0, vs[12])
            for k in range(13, 25):
                sb = sb + vs[k]
            acc = jnp.where(lane_is[2 * u], lanesum(sa), acc)
            acc = jnp.where(lane_is[2 * u + 1], lanesum(sb), acc)
        accv[pl.ds(g * 16, 16)] = 1.0 / (1.0 + jnp.exp(-acc))

    pltpu.sync_copy(accv, out_hbm.at[w])


@functools.partial(
    pl.kernel,
    mesh=plsc.VectorSubcoreMesh(core_axis_name="c", subcore_axis_name="s"),
    out_type=jax.ShapeDtypeStruct((_NW, _BPW), jnp.float32),
    scratch_types=[
        pltpu.VMEM((_IPW,), jnp.int32),
        pltpu.VMEM((_IPW,), jnp.float32),
        pltpu.VMEM((_BPW,), jnp.float32),
        pltpu.SemaphoreType.DMA,
    ],
)
def _pool_kernel(x_hbm, p_hbm, out_hbm, xv, vals, accv, sem):
    _pool_body(x_hbm, p_hbm, out_hbm, xv, vals, accv, sem)


def kernel(x, table, W, b):
    p = _project_table(table, W, b)
    xr = x.reshape(_NW, _IPW)
    out = _pool_kernel(xr, p)
    return out.reshape(BATCH, 1)


# speedup vs baseline: 13.0945x; 1.0014x over previous
import functools

import jax
import jax.numpy as jnp
from jax import lax
from jax.experimental import pallas as pl
from jax.experimental.pallas import tpu as pltpu
from jax.experimental.pallas import tpu_sc as plsc

VOCAB = 1000000
EMBED_DIM = 32
BATCH = 4096
SEQ = 200


_ROWS_PER_BLOCK = 65536
_NPBLK = 16


def _proj_body(t_ref, w_ref, bias_ref, o_ref):
    t = t_ref[...]
    s = jnp.dot(w_ref[...], t,
                preferred_element_type=jnp.float32)[0]
    o_ref[...] = (s + bias_ref[0, 0]) * (1.0 / SEQ)


def _project_table(table, W, b):
    t_t = table.T
    nblk = _NPBLK
    bias = b.reshape(1, 1)
    out = pl.pallas_call(
        _proj_body,
        grid=(nblk,),
        in_specs=[
            pl.BlockSpec((EMBED_DIM, _ROWS_PER_BLOCK), lambda i: (0, i)),
            pl.BlockSpec((1, EMBED_DIM), lambda i: (0, 0)),
            pl.BlockSpec(memory_space=pltpu.SMEM),
        ],
        out_specs=pl.BlockSpec((_ROWS_PER_BLOCK,), lambda i: (i,)),
        out_shape=jax.ShapeDtypeStruct((nblk * _ROWS_PER_BLOCK,),
                                       jnp.float32),
    )(t_t, W, bias)
    return out


_NC, _NS = 2, 16
_NW = _NC * _NS
_BPW = BATCH // _NW
_IPW = _BPW * SEQ


_PAIR = 2 * SEQ
_WROWS = 16
_NWAVE = _BPW // _WROWS
_WELEMS = _WROWS * SEQ


def _pool_body(x_hbm, p_hbm, out_hbm, xv, vals, accv, sem):
    w = lax.axis_index("s") * _NC + lax.axis_index("c")
    pltpu.sync_copy(x_hbm.at[w], xv)

    def fire(g):
        for u in range(_NWAVE):
            off = g * _WELEMS + u * _PAIR
            pltpu.async_copy(
                p_hbm.at[xv.at[pl.ds(off, _PAIR)]],
                vals.at[pl.ds(off, _PAIR)],
                sem,
            )

    def drain(g):
        for u in range(_NWAVE):
            off = g * _WELEMS + u * _PAIR
            pltpu.make_async_copy(
                p_hbm.at[xv.at[pl.ds(off, _PAIR)]],
                vals.at[pl.ds(off, _PAIR)],
                sem,
            ).wait()

    lane = lax.iota(jnp.int32, 16)
    first_half = lane < 8
    lane_is = [lane == i for i in range(_WROWS)]
    bfly = [(lane ^ s)[:, None] for s in (8, 4, 2, 1)]
    _gd = lax.GatherDimensionNumbers(
        offset_dims=(), collapsed_slice_dims=(0,), start_index_map=(0,))

    def lanesum(v):
        for perm in bfly:
            v = v + lax.gather(v, perm, _gd, slice_sizes=(1,),
                               mode=lax.GatherScatterMode.PROMISE_IN_BOUNDS)
        return v

    fire(0)
    for g in range(_NWAVE):
        if g + 1 < _NWAVE:
            fire(g + 1)
        drain(g)
        acc = jnp.zeros((16,), jnp.float32)
        for u in range(_NWAVE):
            base = g * _WELEMS + u * _PAIR
            vs = [vals[pl.ds(base + 16 * k, 16)] for k in range(25)]
            sa = vs[0]
            for k in range(1, 12):
                sa = sa + vs[k]
            sa = sa + jnp.where(first_half, vs[12], 0.0)
            sb = jnp.where(first_half, 0.0, vs[12])
            for k in range(13, 25):
                sb = sb + vs[k]
            acc = jnp.where(lane_is[2 * u], lanesum(sa), acc)
            acc = jnp.where(lane_is[2 * u + 1], lanesum(sb), acc)
        accv[pl.ds(g * 16, 16)] = 1.0 / (1.0 + jnp.exp(-acc))

    pltpu.sync_copy(accv, out_hbm.at[w])


@functools.partial(
    pl.kernel,
    mesh=plsc.VectorSubcoreMesh(core_axis_name="c", subcore_axis_name="s"),
    out_type=jax.ShapeDtypeStruct((_NW, _BPW), jnp.float32),
    scratch_types=[
        pltpu.VMEM((_IPW,), jnp.int32),
        pltpu.VMEM((_IPW,), jnp.float32),
        pltpu.VMEM((_BPW,), jnp.float32),
        pltpu.SemaphoreType.DMA,
    ],
)
def _pool_kernel(x_hbm, p_hbm, out_hbm, xv, vals, accv, sem):
    _pool_body(x_hbm, p_hbm, out_hbm, xv, vals, accv, sem)


def kernel(x, table, W, b):
    p = _project_table(table, W, b)
    xr = x.reshape(_NW, _IPW)
    out = _pool_kernel(xr, p)
    return out.reshape(BATCH, 1)
